# baseline (speedup 1.0000x reference)
import jax
import jax.numpy as jnp
from jax.experimental import pallas as pl
from jax.experimental.pallas import tpu as pltpu


def _logsoftmax_body(x_ref, o_ref):
    x = x_ref[...]
    m = jnp.max(x, axis=1, keepdims=True)
    s = jnp.log(jnp.sum(jnp.exp(x - m), axis=1, keepdims=True))
    o_ref[...] = x - m - s


def _gcn_norm(edge_index, num_nodes):
    row, col = edge_index[0], edge_index[1]
    loop = jnp.arange(num_nodes, dtype=edge_index.dtype)
    row = jnp.concatenate([row, loop])
    col = jnp.concatenate([col, loop])
    w = jnp.ones((row.shape[0],), dtype=jnp.float32)
    deg = jnp.zeros((num_nodes,), dtype=jnp.float32).at[row].add(w)
    dinv = jnp.where(deg > 0, 1.0 / jnp.sqrt(deg), 0.0)
    norm = dinv[row] * w * dinv[col]
    return jnp.stack([row, col]), norm


def _mask_features(x, edge_index, edge_weight, sigma):
    src, tgt = edge_index[0], edge_index[1]
    h_s = x[src]
    h_t = x[tgt]
    h = (h_t - h_s) / sigma
    h = edge_weight[:, None] * h * h
    mask = jnp.zeros_like(x).at[src].add(h)
    deg = jnp.bincount(src, length=x.shape[0]).astype(x.dtype)
    mask = jnp.exp(-mask / deg[:, None])
    return x * mask


def _conv(x, edge_index, norm, sigma, W, b):
    x = _mask_features(x, edge_index, norm, sigma)
    x = x @ W
    row, col = edge_index[0], edge_index[1]
    out = jnp.zeros((x.shape[0], x.shape[1]), dtype=x.dtype).at[row].add(norm[:, None] * x[col])
    return out + b


def kernel(x, edge_index, sigma1, W1, b1, sigma2, W2, b2):
    ei, norm = _gcn_norm(edge_index, x.shape[0])
    h = _conv(x, ei, norm, sigma1, W1, b1)
    h = jax.nn.relu(h)
    h = _conv(h, ei, norm, sigma2, W2, b2)
    out = pl.pallas_call(
        _logsoftmax_body,
        out_shape=jax.ShapeDtypeStruct(h.shape, h.dtype),
    )(h)
    return out


# speedup vs baseline: 5.6618x; 5.6618x over previous
import functools

import jax
import jax.numpy as jnp
from jax import lax
from jax.experimental import pallas as pl
from jax.experimental.pallas import tpu as pltpu
from jax.experimental.pallas import tpu_sc as plsc

N = 10000
E = 320000
F = 128
NCLS = 40
NC = 2
NS = 16
B = 128
NW = NC * NS

E_PAD = 327680
N_ACC = 10112
RPT = N_ACC // NS

_ROWBLK = 1000
_GRID = N // _ROWBLK

_SC_PARAMS = pltpu.CompilerParams(use_tc_tiling_on_sc=False)


def _mesh():
    return plsc.VectorSubcoreMesh(
        core_axis_name="c", subcore_axis_name="s", num_cores=NC, num_subcores=NS)


_NB_DEG = E_PAD // (NW * B)


def _sc_degree_body(row2d, ones_hbm, zeros_hbm, out_hbm,
                    rowv, onesv, stage, acc, sem):
    c = lax.axis_index("c")
    s = lax.axis_index("s")
    wid = s * NC + c
    pltpu.sync_copy(row2d.at[pl.ds(wid * _NB_DEG, _NB_DEG), :], rowv)
    pltpu.sync_copy(ones_hbm, onesv)
    pltpu.sync_copy(zeros_hbm.at[pl.ds(s * RPT, RPT), :], stage)
    pltpu.sync_copy(stage, acc.at[pl.ds(s * RPT, RPT), :])
    plsc.subcore_barrier()

    def body(j, carry):
        pltpu.sync_copy(onesv, acc.at[rowv.at[j]], add=True)
        return carry

    lax.fori_loop(0, _NB_DEG, body, 0)
    plsc.subcore_barrier()
    pltpu.sync_copy(acc.at[pl.ds(s * RPT, RPT), :], stage)
    pltpu.sync_copy(stage, out_hbm.at[c, pl.ds(s * RPT, RPT), :])


def _sc_degree(row2d):
    ones = jnp.ones((B, 16), jnp.float32)
    zeros = jnp.zeros((N_ACC, 16), jnp.float32)
    k = pl.kernel(
        _sc_degree_body,
        out_type=jax.ShapeDtypeStruct((NC, N_ACC, 16), jnp.float32),
        mesh=_mesh(),
        scratch_types=[
            pltpu.VMEM((_NB_DEG, B), jnp.int32),
            pltpu.VMEM((B, 16), jnp.float32),
            pltpu.VMEM((RPT, 16), jnp.float32),
            pltpu.VMEM_SHARED((N_ACC, 16), jnp.float32),
            pltpu.SemaphoreType.DMA,
        ],
        compiler_params=_SC_PARAMS,
    )
    return k(row2d, ones, zeros)


def _zero_acc(s, zeros_hbm, buf0, acc):
    pltpu.sync_copy(zeros_hbm, buf0)
    off = 0
    while off < RPT:
        rows = min(B, RPT - off)
        pltpu.sync_copy(buf0.at[pl.ds(0, rows), :],
                        acc.at[pl.ds(s * RPT + off, rows), :])
        off += rows


def _write_out(c, s, buf0, acc, out_hbm):
    off = 0
    while off < RPT:
        rows = min(B, RPT - off)
        pltpu.sync_copy(acc.at[pl.ds(s * RPT + off, rows), :],
                        buf0.at[pl.ds(0, rows), :])
        pltpu.sync_copy(buf0.at[pl.ds(0, rows), :],
                        out_hbm.at[c, pl.ds(s * RPT + off, rows), :])
        off += rows


def _scatter_loop(table, nb, rowv, colv, buf0, buf1, acc, sem0, sem1):
    pltpu.async_copy(table.at[colv.at[0]], buf0, sem0)

    def body(j2, carry):
        j = 2 * j2
        pltpu.async_copy(table.at[colv.at[j + 1]], buf1, sem1)
        pltpu.make_async_copy(table.at[colv.at[j]], buf0, sem0).wait()
        pltpu.sync_copy(buf0, acc.at[rowv.at[j]], add=True)

        @pl.when(j + 2 < nb)
        def _():
            pltpu.async_copy(table.at[colv.at[j + 2]], buf0, sem0)

        pltpu.make_async_copy(table.at[colv.at[j + 1]], buf1, sem1).wait()
        pltpu.sync_copy(buf1, acc.at[rowv.at[j + 1]], add=True)
        return carry

    lax.fori_loop(0, nb // 2, body, 0)


def _scatter_body_fs(nb, t0, t1, row2d, col2d, zeros_hbm, out_hbm,
                     rowv, colv, buf0, buf1, acc, sem0, sem1):
    c = lax.axis_index("c")
    s = lax.axis_index("s")
    pltpu.sync_copy(row2d.at[pl.ds(s * nb, nb), :], rowv)
    pltpu.sync_copy(col2d.at[pl.ds(s * nb, nb), :], colv)
    _zero_acc(s, zeros_hbm, buf0, acc)
    plsc.subcore_barrier()

    @pl.when(c == 0)
    def _():
        _scatter_loop(t0, nb, rowv, colv, buf0, buf1, acc, sem0, sem1)

    @pl.when(c == 1)
    def _():
        _scatter_loop(t1, nb, rowv, colv, buf0, buf1, acc, sem0, sem1)

    plsc.subcore_barrier()
    _write_out(c, s, buf0, acc, out_hbm)


def _scatter_body_es(nb, t0, row2d, col2d, zeros_hbm, out_hbm,
                     rowv, colv, buf0, buf1, acc, sem0, sem1):
    c = lax.axis_index("c")
    s = lax.axis_index("s")
    wid = s * NC + c
    pltpu.sync_copy(row2d.at[pl.ds(wid * nb, nb), :], rowv)
    pltpu.sync_copy(col2d.at[pl.ds(wid * nb, nb), :], colv)
    _zero_acc(s, zeros_hbm, buf0, acc)
    plsc.subcore_barrier()
    _scatter_loop(t0, nb, rowv, colv, buf0, buf1, acc, sem0, sem1)
    plsc.subcore_barrier()
    _write_out(c, s, buf0, acc, out_hbm)


def _scatter_kernel(ncols, feature_split):
    nb = E_PAD // ((NS if feature_split else NW) * B)
    body = functools.partial(
        _scatter_body_fs if feature_split else _scatter_body_es, nb)
    n_tables = 2 if feature_split else 1
    return pl.kernel(
        body,
        out_type=jax.ShapeDtypeStruct((NC, N_ACC, ncols), jnp.float32),
        mesh=_mesh(),
        scratch_types=[
            pltpu.VMEM((nb, B), jnp.int32),
            pltpu.VMEM((nb, B), jnp.int32),
            pltpu.VMEM((B, ncols), jnp.float32),
            pltpu.VMEM((B, ncols), jnp.float32),
            pltpu.VMEM_SHARED((N_ACC, ncols), jnp.float32),
            pltpu.SemaphoreType.DMA,
            pltpu.SemaphoreType.DMA,
        ],
        compiler_params=_SC_PARAMS,
    ), n_tables


def _scan_scatter_fs(t0s, t1s, row2d, col2d, ncols):
    kern, _ = _scatter_kernel(ncols, feature_split=True)
    zeros = jnp.zeros((B, ncols), jnp.float32)

    def step(carry, ts):
        return carry, kern(ts[0], ts[1], row2d, col2d, zeros)

    _, outs = lax.scan(step, 0, (t0s, t1s))
    return outs


def _scan_scatter_es(ts, row2d, col2d, ncols):
    kern, _ = _scatter_kernel(ncols, feature_split=False)
    zeros = jnp.zeros((B, ncols), jnp.float32)

    def step(carry, t):
        return carry, kern(t, row2d, col2d, zeros)

    _, outs = lax.scan(step, 0, ts)
    return outs


def _dinv_of(pa_ref, pb_ref):
    deg = pa_ref[0, :, 0:1] + pb_ref[0, :, 0:1] + 1.0
    return lax.rsqrt(deg)


_DEG_SPEC_A = pl.BlockSpec((1, _ROWBLK, 16), lambda i: (0, i, 0))
_DEG_SPEC_B = pl.BlockSpec((1, _ROWBLK, 16), lambda i: (1, i, 0))


def _row_spec(ncols):
    return pl.BlockSpec((_ROWBLK, ncols), lambda i: (i, 0))


def _stack_spec(k, ncols):
    return pl.BlockSpec((1, _ROWBLK, ncols),
                        functools.partial(lambda k_, i: (k_, i, 0), k))


def _piece_spec(k, c, ncols):
    return pl.BlockSpec((1, 1, _ROWBLK, ncols),
                        functools.partial(lambda k_, c_, i: (k_, c_, i, 0), k, c))


def _full_spec(r, c):
    return pl.BlockSpec((r, c), lambda i: (0, 0))


def _tc_tables1_body(pa, pb, x_ref, t0_ref, t1_ref):
    dinv = _dinv_of(pa, pb)
    x = x_ref[...]
    u1 = dinv * x
    u2 = u1 * x
    dinv16 = dinv + jnp.zeros((_ROWBLK, 16), jnp.float32)

    def chunk(base, k):
        lo = base + 32 * k
        return jnp.concatenate([u1[:, lo:lo + 32], u2[:, lo:lo + 32], dinv16],
                               axis=1)

    t0_ref[...] = jnp.stack([chunk(0, 0), chunk(0, 1)])
    t1_ref[...] = jnp.stack([chunk(64, 0), chunk(64, 1)])


def _tc_tables1(deg_parts, x):
    return pl.pallas_call(
        _tc_tables1_body,
        grid=(_GRID,),
        in_specs=[_DEG_SPEC_A, _DEG_SPEC_B, _row_spec(F)],
        out_specs=[pl.BlockSpec((2, _ROWBLK, 80), lambda i: (0, i, 0))] * 2,
        out_shape=[jax.ShapeDtypeStruct((2, N, 80), jnp.float32)] * 2,
    )(deg_parts, deg_parts, x)


def _tc_mask_mm_body(sig2_inv_ref, w_ref, pa, pb, x_ref,
                     p00, p01, p10, p11, out_ref):
    dinv = _dinv_of(pa, pb)
    x = x_ref[...]
    s1 = jnp.concatenate([p00[0, 0, :, 0:32], p10[0, 0, :, 0:32],
                          p01[0, 0, :, 0:32], p11[0, 0, :, 0:32]], axis=1)
    s2 = jnp.concatenate([p00[0, 0, :, 32:64], p10[0, 0, :, 32:64],
                          p01[0, 0, :, 32:64], p11[0, 0, :, 32:64]], axis=1)
    s0 = p00[0, 0, :, 64:65]
    bracket = s2 - 2.0 * x * s1 + x * x * s0
    mask = jnp.exp(-(dinv * dinv * dinv) * bracket * sig2_inv_ref[...])
    y = jnp.dot(x * mask, w_ref[...], preferred_element_type=jnp.float32)
    dy = dinv * y
    out_ref[...] = jnp.stack([dy[:, 32 * k:32 * k + 32] for k in range(4)])


def _tc_mask_mm(sig2_inv, w, deg_parts, x, sp1):
    return pl.pallas_call(
        _tc_mask_mm_body,
        grid=(_GRID,),
        in_specs=[_full_spec(1, F), _full_spec(F, F),
                  _DEG_SPEC_A, _DEG_SPEC_B, _row_spec(F),
                  _piece_spec(0, 0, 80), _piece_spec(0, 1, 80),
                  _piece_spec(1, 0, 80), _piece_spec(1, 1, 80)],
        out_specs=pl.BlockSpec((4, _ROWBLK, 32), lambda i: (0, i, 0)),
        out_shape=jax.ShapeDtypeStruct((4, N, 32), jnp.float32),
    )(sig2_inv, w, deg_parts, deg_parts, x, sp1, sp1, sp1, sp1)


def _tc_relu_tables_body(b1_ref, pa, pb, z00, z01, z10, z11, z20, z21,
                         z30, z31, y1p_ref, h_ref, t0_ref, t1_ref):
    dinv = _dinv_of(pa, pb)
    zs = [z00[0, 0] + z01[0, 0], z10[0, 0] + z11[0, 0],
          z20[0, 0] + z21[0, 0], z30[0, 0] + z31[0, 0]]
    z = jnp.concatenate(zs, axis=1)
    y1p = jnp.concatenate([y1p_ref[k] for k in range(4)], axis=1)
    h = dinv * (z + y1p) + b1_ref[...]
    h = jnp.maximum(h, 0.0)
    h_ref[...] = h
    u1 = dinv * h
    u2 = u1 * h

    def chunk(base, k):
        lo = base + 16 * k
        return jnp.concatenate([u1[:, lo:lo + 16], u2[:, lo:lo + 16]], axis=1)

    t0_ref[...] = jnp.stack([chunk(0, k) for k in range(4)])
    t1_ref[...] = jnp.stack([chunk(64, k) for k in range(4)])


def _tc_relu_tables(b1, deg_parts, zp2, y1ps):
    return pl.pallas_call(
        _tc_relu_tables_body,
        grid=(_GRID,),
        in_specs=[_full_spec(1, F), _DEG_SPEC_A, _DEG_SPEC_B]
        + [_piece_spec(k, c, 32) for k in range(4) for c in range(2)]
        + [pl.BlockSpec((4, _ROWBLK, 32), lambda i: (0, i, 0))],
        out_specs=[_row_spec(F)]
        + [pl.BlockSpec((4, _ROWBLK, 32), lambda i: (0, i, 0))] * 2,
        out_shape=[jax.ShapeDtypeStruct((N, F), jnp.float32)]
        + [jax.ShapeDtypeStruct((4, N, 32), jnp.float32)] * 2,
    )(b1, deg_parts, deg_parts, *([zp2] * 8), y1ps)


def _tc_mask_mm2_body(sig2_inv_ref, w_ref, pa, pb, h_ref, s0p,
                      t00, t01, t10, t11, t20, t21, t30, t31, out_ref):
    dinv = _dinv_of(pa, pb)
    h = h_ref[...]
    ts = [t00, t10, t20, t30, t01, t11, t21, t31]
    s1 = jnp.concatenate([t[0, 0, :, 0:16] for t in ts], axis=1)
    s2 = jnp.concatenate([t[0, 0, :, 16:32] for t in ts], axis=1)
    s0 = s0p[0, 0, :, 64:65]
    bracket = s2 - 2.0 * h * s1 + h * h * s0
    mask = jnp.exp(-(dinv * dinv * dinv) * bracket * sig2_inv_ref[...])
    y = jnp.dot(h * mask, w_ref[...], preferred_element_type=jnp.float32)
    dy = dinv * y
    out_ref[...] = jnp.stack([dy[:, 0:32], dy[:, 32:64]])


def _tc_mask_mm2(sig2_inv, wpad, deg_parts, h, sp3, sp1):
    return pl.pallas_call(
        _tc_mask_mm2_body,
        grid=(_GRID,),
        in_specs=[_full_spec(1, F), _full_spec(F, 64),
                  _DEG_SPEC_A, _DEG_SPEC_B, _row_spec(F),
                  _piece_spec(0, 0, 80)]
        + [_piece_spec(k, c, 32) for k in range(4) for c in range(2)],
        out_specs=pl.BlockSpec((2, _ROWBLK, 32), lambda i: (0, i, 0)),
        out_shape=jax.ShapeDtypeStruct((2, N, 32), jnp.float32),
    )(sig2_inv, wpad, deg_parts, deg_parts, h, sp1, *([sp3] * 8))


def _tc_final_body(b2_ref, pa, pb, z00, z01, z10, z11, y2p_ref, out_ref):
    dinv = _dinv_of(pa, pb)
    zs = [z00[0, 0] + z01[0, 0], z10[0, 0] + z11[0, 0]]
    z = jnp.concatenate(zs, axis=1)
    y2p = jnp.concatenate([y2p_ref[0], y2p_ref[1]], axis=1)
    logits = dinv * (z + y2p) + b2_ref[...]
    colid = lax.broadcasted_iota(jnp.int32, (_ROWBLK, 64), 1)
    valid = colid < NCLS
    neg = jnp.full_like(logits, -jnp.inf)
    m = jnp.max(jnp.where(valid, logits, neg), axis=1, keepdims=True)
    e = jnp.where(valid, jnp.exp(logits - m), 0.0)
    lse = jnp.log(jnp.sum(e, axis=1, keepdims=True))
    out_ref[...] = logits - m - lse


def _tc_final(b2pad, deg_parts, zp4, y2ps):
    return pl.pallas_call(
        _tc_final_body,
        grid=(_GRID,),
        in_specs=[_full_spec(1, 64), _DEG_SPEC_A, _DEG_SPEC_B]
        + [_piece_spec(k, c, 32) for k in range(2) for c in range(2)]
        + [pl.BlockSpec((2, _ROWBLK, 32), lambda i: (0, i, 0))],
        out_specs=_row_spec(64),
        out_shape=jax.ShapeDtypeStruct((N, 64), jnp.float32),
    )(b2pad, deg_parts, deg_parts, *([zp4] * 4), y2ps)


def kernel(x, edge_index, sigma1, W1, b1, sigma2, W2, b2):
    row = edge_index[0]
    col = edge_index[1]
    pad_r = jnp.full((E_PAD - E,), N, jnp.int32)
    pad_c = jnp.zeros((E_PAD - E,), jnp.int32)
    row2d = jnp.concatenate([row, pad_r]).reshape(E_PAD // B, B)
    col2d = jnp.concatenate([col, pad_c]).reshape(E_PAD // B, B)

    deg_parts = _sc_degree(row2d)

    sig1_inv = 1.0 / (sigma1 * sigma1)
    sig2_inv = 1.0 / (sigma2 * sigma2)
    b1r = b1.reshape(1, F)
    b2pad = jnp.pad(b2, (0, 64 - NCLS)).reshape(1, 64)
    w2pad = jnp.pad(W2, ((0, 0), (0, 64 - NCLS)))

    t0s, t1s = _tc_tables1(deg_parts, x)
    sp1 = _scan_scatter_fs(t0s, t1s, row2d, col2d, 80)
    y1ps = _tc_mask_mm(sig1_inv, W1, deg_parts, x, sp1)
    zp2 = _scan_scatter_es(y1ps, row2d, col2d, 32)
    h, t0bs, t1bs = _tc_relu_tables(b1r, deg_parts, zp2, y1ps)

    sp3 = _scan_scatter_fs(t0bs, t1bs, row2d, col2d, 32)
    y2ps = _tc_mask_mm2(sig2_inv, w2pad, deg_parts, h, sp3, sp1)
    zp4 = _scan_scatter_es(y2ps, row2d, col2d, 32)
    out = _tc_final(b2pad, deg_parts, zp4, y2ps)
    return out[:, :NCLS]


# speedup vs baseline: 5.6650x; 1.0006x over previous
import functools

import jax
import jax.numpy as jnp
from jax import lax
from jax.experimental import pallas as pl
from jax.experimental.pallas import tpu as pltpu
from jax.experimental.pallas import tpu_sc as plsc

N = 10000
E = 320000
F = 128
NCLS = 40
NC = 2
NS = 16
B = 128
NW = NC * NS

E_PAD = 327680
N_ACC = 10112
RPT = N_ACC // NS

_ROWBLK = 1000
_GRID = N // _ROWBLK

_SC_PARAMS = pltpu.CompilerParams(use_tc_tiling_on_sc=False)


def _mesh():
    return plsc.VectorSubcoreMesh(
        core_axis_name="c", subcore_axis_name="s", num_cores=NC, num_subcores=NS)


_NB_DEG = E_PAD // (NW * B)


_DEGC = 8


def _sc_degree_body(row2d, ones_hbm, zeros_hbm, out_hbm,
                    rowv, onesv, stage, acc, sem):
    c = lax.axis_index("c")
    s = lax.axis_index("s")
    wid = s * NC + c
    pltpu.sync_copy(row2d.at[pl.ds(wid * _NB_DEG, _NB_DEG), :], rowv)
    pltpu.sync_copy(ones_hbm, onesv)
    pltpu.sync_copy(zeros_hbm.at[pl.ds(s * RPT, RPT), :], stage)
    pltpu.sync_copy(stage, acc.at[pl.ds(s * RPT, RPT), :])
    plsc.subcore_barrier()

    def body(j, carry):
        pltpu.sync_copy(onesv, acc.at[rowv.at[j]], add=True)
        return carry

    lax.fori_loop(0, _NB_DEG, body, 0)
    plsc.subcore_barrier()
    pltpu.sync_copy(acc.at[pl.ds(s * RPT, RPT), :], stage)
    pltpu.sync_copy(stage, out_hbm.at[c, pl.ds(s * RPT, RPT), :])


def _sc_degree(row2d):
    ones = jnp.ones((B, _DEGC), jnp.float32)
    zeros = jnp.zeros((N_ACC, _DEGC), jnp.float32)
    k = pl.kernel(
        _sc_degree_body,
        out_type=jax.ShapeDtypeStruct((NC, N_ACC, _DEGC), jnp.float32),
        mesh=_mesh(),
        scratch_types=[
            pltpu.VMEM((_NB_DEG, B), jnp.int32),
            pltpu.VMEM((B, _DEGC), jnp.float32),
            pltpu.VMEM((RPT, _DEGC), jnp.float32),
            pltpu.VMEM_SHARED((N_ACC, _DEGC), jnp.float32),
            pltpu.SemaphoreType.DMA,
        ],
        compiler_params=_SC_PARAMS,
    )
    return k(row2d, ones, zeros)


def _zero_acc(s, zeros_hbm, buf0, acc):
    pltpu.sync_copy(zeros_hbm, buf0)
    off = 0
    while off < RPT:
        rows = min(B, RPT - off)
        pltpu.sync_copy(buf0.at[pl.ds(0, rows), :],
                        acc.at[pl.ds(s * RPT + off, rows), :])
        off += rows


def _write_out(c, s, buf0, acc, out_hbm):
    off = 0
    while off < RPT:
        rows = min(B, RPT - off)
        pltpu.sync_copy(acc.at[pl.ds(s * RPT + off, rows), :],
                        buf0.at[pl.ds(0, rows), :])
        pltpu.sync_copy(buf0.at[pl.ds(0, rows), :],
                        out_hbm.at[c, pl.ds(s * RPT + off, rows), :])
        off += rows


_NBUF = 4


def _scatter_loop(table, nb, rowv, colv, bufs, acc, gsems, ssems):
    for b in range(_NBUF):
        pltpu.async_copy(table.at[colv.at[b]], bufs[b], gsems[b])

    def body(j4, carry):
        base = _NBUF * j4
        for b in range(_NBUF):
            pltpu.make_async_copy(table.at[colv.at[base + b]],
                                  bufs[b], gsems[b]).wait()
            pltpu.async_copy(bufs[b], acc.at[rowv.at[base + b]],
                             ssems[b], add=True)
        for b in range(_NBUF):
            nxt = base + _NBUF + b

            @pl.when(nxt < nb)
            def _(b=b, nxt=nxt):
                pltpu.make_async_copy(bufs[b], acc.at[rowv.at[nxt - _NBUF]],
                                      ssems[b]).wait()
                pltpu.async_copy(table.at[colv.at[nxt]], bufs[b], gsems[b])
        return carry

    lax.fori_loop(0, nb // _NBUF, body, 0)
    for b in range(_NBUF):
        pltpu.make_async_copy(bufs[b], acc.at[rowv.at[nb - _NBUF + b]],
                              ssems[b]).wait()


def _scatter_body_fs(nb, t0, t1, row2d, col2d, zeros_hbm, out_hbm,
                     rowv, colv, b0, b1, b2, b3, acc,
                     g0, g1, g2, g3, s0, s1, s2, s3):
    c = lax.axis_index("c")
    s = lax.axis_index("s")
    bufs, gsems, ssems = (b0, b1, b2, b3), (g0, g1, g2, g3), (s0, s1, s2, s3)
    pltpu.sync_copy(row2d.at[pl.ds(s * nb, nb), :], rowv)
    pltpu.sync_copy(col2d.at[pl.ds(s * nb, nb), :], colv)
    _zero_acc(s, zeros_hbm, b0, acc)
    plsc.subcore_barrier()

    @pl.when(c == 0)
    def _():
        _scatter_loop(t0, nb, rowv, colv, bufs, acc, gsems, ssems)

    @pl.when(c == 1)
    def _():
        _scatter_loop(t1, nb, rowv, colv, bufs, acc, gsems, ssems)

    plsc.subcore_barrier()
    _write_out(c, s, b0, acc, out_hbm)


def _scatter_body_es(nb, t0, row2d, col2d, zeros_hbm, out_hbm,
                     rowv, colv, b0, b1, b2, b3, acc,
                     g0, g1, g2, g3, s0, s1, s2, s3):
    c = lax.axis_index("c")
    s = lax.axis_index("s")
    bufs, gsems, ssems = (b0, b1, b2, b3), (g0, g1, g2, g3), (s0, s1, s2, s3)
    wid = s * NC + c
    pltpu.sync_copy(row2d.at[pl.ds(wid * nb, nb), :], rowv)
    pltpu.sync_copy(col2d.at[pl.ds(wid * nb, nb), :], colv)
    _zero_acc(s, zeros_hbm, b0, acc)
    plsc.subcore_barrier()
    _scatter_loop(t0, nb, rowv, colv, bufs, acc, gsems, ssems)
    plsc.subcore_barrier()
    _write_out(c, s, b0, acc, out_hbm)


def _scatter_kernel(ncols, feature_split):
    nb = E_PAD // ((NS if feature_split else NW) * B)
    body = functools.partial(
        _scatter_body_fs if feature_split else _scatter_body_es, nb)
    return pl.kernel(
        body,
        out_type=jax.ShapeDtypeStruct((NC, N_ACC, ncols), jnp.float32),
        mesh=_mesh(),
        scratch_types=[
            pltpu.VMEM((nb, B), jnp.int32),
            pltpu.VMEM((nb, B), jnp.int32),
        ]
        + [pltpu.VMEM((B, ncols), jnp.float32)] * _NBUF
        + [pltpu.VMEM_SHARED((N_ACC, ncols), jnp.float32)]
        + [pltpu.SemaphoreType.DMA] * (2 * _NBUF),
        compiler_params=_SC_PARAMS,
    ), None


def _scan_scatter_fs(t0s, t1s, row2d, col2d, ncols):
    kern, _ = _scatter_kernel(ncols, feature_split=True)
    zeros = jnp.zeros((B, ncols), jnp.float32)

    def step(carry, ts):
        return carry, kern(ts[0], ts[1], row2d, col2d, zeros)

    _, outs = lax.scan(step, 0, (t0s, t1s))
    return outs


def _scan_scatter_es(ts, row2d, col2d, ncols):
    kern, _ = _scatter_kernel(ncols, feature_split=False)
    zeros = jnp.zeros((B, ncols), jnp.float32)

    def step(carry, t):
        return carry, kern(t, row2d, col2d, zeros)

    _, outs = lax.scan(step, 0, ts)
    return outs


def _dinv_of(pa_ref, pb_ref):
    deg = pa_ref[0, :, 0:1] + pb_ref[0, :, 0:1] + 1.0
    return lax.rsqrt(deg)


_DEG_SPEC_A = pl.BlockSpec((1, _ROWBLK, _DEGC), lambda i: (0, i, 0))
_DEG_SPEC_B = pl.BlockSpec((1, _ROWBLK, _DEGC), lambda i: (1, i, 0))


def _row_spec(ncols):
    return pl.BlockSpec((_ROWBLK, ncols), lambda i: (i, 0))


def _stack_spec(k, ncols):
    return pl.BlockSpec((1, _ROWBLK, ncols),
                        functools.partial(lambda k_, i: (k_, i, 0), k))


def _piece_spec(k, c, ncols):
    return pl.BlockSpec((1, 1, _ROWBLK, ncols),
                        functools.partial(lambda k_, c_, i: (k_, c_, i, 0), k, c))


def _full_spec(r, c):
    return pl.BlockSpec((r, c), lambda i: (0, 0))


def _tc_tables1_body(pa, pb, x_ref, t0_ref, t1_ref):
    dinv = _dinv_of(pa, pb)
    x = x_ref[...]
    u1 = dinv * x
    u2 = u1 * x
    dinv4 = dinv + jnp.zeros((_ROWBLK, 4), jnp.float32)

    def chunk(base, k):
        lo = base + 32 * k
        return jnp.concatenate([u1[:, lo:lo + 32], u2[:, lo:lo + 32], dinv4],
                               axis=1)

    t0_ref[...] = jnp.stack([chunk(0, 0), chunk(0, 1)])
    t1_ref[...] = jnp.stack([chunk(64, 0), chunk(64, 1)])


def _tc_tables1(deg_parts, x):
    return pl.pallas_call(
        _tc_tables1_body,
        grid=(_GRID,),
        in_specs=[_DEG_SPEC_A, _DEG_SPEC_B, _row_spec(F)],
        out_specs=[pl.BlockSpec((2, _ROWBLK, 68), lambda i: (0, i, 0))] * 2,
        out_shape=[jax.ShapeDtypeStruct((2, N, 68), jnp.float32)] * 2,
    )(deg_parts, deg_parts, x)


def _tc_mask_mm_body(sig2_inv_ref, w_ref, pa, pb, x_ref,
                     p00, p01, p10, p11, out_ref):
    dinv = _dinv_of(pa, pb)
    x = x_ref[...]
    s1 = jnp.concatenate([p00[0, 0, :, 0:32], p10[0, 0, :, 0:32],
                          p01[0, 0, :, 0:32], p11[0, 0, :, 0:32]], axis=1)
    s2 = jnp.concatenate([p00[0, 0, :, 32:64], p10[0, 0, :, 32:64],
                          p01[0, 0, :, 32:64], p11[0, 0, :, 32:64]], axis=1)
    s0 = p00[0, 0, :, 64:65]
    bracket = s2 - 2.0 * x * s1 + x * x * s0
    mask = jnp.exp(-(dinv * dinv * dinv) * bracket * sig2_inv_ref[...])
    y = jnp.dot(x * mask, w_ref[...], preferred_element_type=jnp.float32)
    dy = dinv * y
    out_ref[...] = jnp.stack([dy[:, 32 * k:32 * k + 32] for k in range(4)])


def _tc_mask_mm(sig2_inv, w, deg_parts, x, sp1):
    return pl.pallas_call(
        _tc_mask_mm_body,
        grid=(_GRID,),
        in_specs=[_full_spec(1, F), _full_spec(F, F),
                  _DEG_SPEC_A, _DEG_SPEC_B, _row_spec(F),
                  _piece_spec(0, 0, 68), _piece_spec(0, 1, 68),
                  _piece_spec(1, 0, 68), _piece_spec(1, 1, 68)],
        out_specs=pl.BlockSpec((4, _ROWBLK, 32), lambda i: (0, i, 0)),
        out_shape=jax.ShapeDtypeStruct((4, N, 32), jnp.float32),
    )(sig2_inv, w, deg_parts, deg_parts, x, sp1, sp1, sp1, sp1)


def _tc_relu_tables_body(b1_ref, pa, pb, z00, z01, z10, z11, z20, z21,
                         z30, z31, y1p_ref, h_ref, t0_ref, t1_ref):
    dinv = _dinv_of(pa, pb)
    zs = [z00[0, 0] + z01[0, 0], z10[0, 0] + z11[0, 0],
          z20[0, 0] + z21[0, 0], z30[0, 0] + z31[0, 0]]
    z = jnp.concatenate(zs, axis=1)
    y1p = jnp.concatenate([y1p_ref[k] for k in range(4)], axis=1)
    h = dinv * (z + y1p) + b1_ref[...]
    h = jnp.maximum(h, 0.0)
    h_ref[...] = h
    u1 = dinv * h
    u2 = u1 * h

    def chunk(base, k):
        lo = base + 32 * k
        return jnp.concatenate([u1[:, lo:lo + 32], u2[:, lo:lo + 32]], axis=1)

    t0_ref[...] = jnp.stack([chunk(0, k) for k in range(2)])
    t1_ref[...] = jnp.stack([chunk(64, k) for k in range(2)])


def _tc_relu_tables(b1, deg_parts, zp2, y1ps):
    return pl.pallas_call(
        _tc_relu_tables_body,
        grid=(_GRID,),
        in_specs=[_full_spec(1, F), _DEG_SPEC_A, _DEG_SPEC_B]
        + [_piece_spec(k, c, 32) for k in range(4) for c in range(2)]
        + [pl.BlockSpec((4, _ROWBLK, 32), lambda i: (0, i, 0))],
        out_specs=[_row_spec(F)]
        + [pl.BlockSpec((2, _ROWBLK, 64), lambda i: (0, i, 0))] * 2,
        out_shape=[jax.ShapeDtypeStruct((N, F), jnp.float32)]
        + [jax.ShapeDtypeStruct((2, N, 64), jnp.float32)] * 2,
    )(b1, deg_parts, deg_parts, *([zp2] * 8), y1ps)


def _tc_mask_mm2_body(sig2_inv_ref, w_ref, pa, pb, h_ref, s0p,
                      t00, t01, t10, t11, out_ref):
    dinv = _dinv_of(pa, pb)
    h = h_ref[...]
    ts = [t00, t10, t01, t11]
    s1 = jnp.concatenate([t[0, 0, :, 0:32] for t in ts], axis=1)
    s2 = jnp.concatenate([t[0, 0, :, 32:64] for t in ts], axis=1)
    s0 = s0p[0, 0, :, 64:65]
    bracket = s2 - 2.0 * h * s1 + h * h * s0
    mask = jnp.exp(-(dinv * dinv * dinv) * bracket * sig2_inv_ref[...])
    y = jnp.dot(h * mask, w_ref[...], preferred_element_type=jnp.float32)
    dy = dinv * y
    out_ref[...] = jnp.stack([dy[:, 0:32], dy[:, 32:64]])


def _tc_mask_mm2(sig2_inv, wpad, deg_parts, h, sp3, sp1):
    return pl.pallas_call(
        _tc_mask_mm2_body,
        grid=(_GRID,),
        in_specs=[_full_spec(1, F), _full_spec(F, 64),
                  _DEG_SPEC_A, _DEG_SPEC_B, _row_spec(F),
                  _piece_spec(0, 0, 68)]
        + [_piece_spec(k, c, 64) for k in range(2) for c in range(2)],
        out_specs=pl.BlockSpec((2, _ROWBLK, 32), lambda i: (0, i, 0)),
        out_shape=jax.ShapeDtypeStruct((2, N, 32), jnp.float32),
    )(sig2_inv, wpad, deg_parts, deg_parts, h, sp1, *([sp3] * 4))


def _tc_final_body(b2_ref, pa, pb, z00, z01, z10, z11, y2p_ref, out_ref):
    dinv = _dinv_of(pa, pb)
    zs = [z00[0, 0] + z01[0, 0], z10[0, 0] + z11[0, 0]]
    z = jnp.concatenate(zs, axis=1)
    y2p = jnp.concatenate([y2p_ref[0], y2p_ref[1]], axis=1)
    logits = dinv * (z + y2p) + b2_ref[...]
    colid = lax.broadcasted_iota(jnp.int32, (_ROWBLK, 64), 1)
    valid = colid < NCLS
    neg = jnp.full_like(logits, -jnp.inf)
    m = jnp.max(jnp.where(valid, logits, neg), axis=1, keepdims=True)
    e = jnp.where(valid, jnp.exp(logits - m), 0.0)
    lse = jnp.log(jnp.sum(e, axis=1, keepdims=True))
    out_ref[...] = logits - m - lse


def _tc_final(b2pad, deg_parts, zp4, y2ps):
    return pl.pallas_call(
        _tc_final_body,
        grid=(_GRID,),
        in_specs=[_full_spec(1, 64), _DEG_SPEC_A, _DEG_SPEC_B]
        + [_piece_spec(k, c, 32) for k in range(2) for c in range(2)]
        + [pl.BlockSpec((2, _ROWBLK, 32), lambda i: (0, i, 0))],
        out_specs=_row_spec(64),
        out_shape=jax.ShapeDtypeStruct((N, 64), jnp.float32),
    )(b2pad, deg_parts, deg_parts, *([zp4] * 4), y2ps)


def kernel(x, edge_index, sigma1, W1, b1, sigma2, W2, b2):
    row = edge_index[0]
    col = edge_index[1]
    pad_r = jnp.full((E_PAD - E,), N, jnp.int32)
    pad_c = jnp.zeros((E_PAD - E,), jnp.int32)
    row2d = jnp.concatenate([row, pad_r]).reshape(E_PAD // B, B)
    col2d = jnp.concatenate([col, pad_c]).reshape(E_PAD // B, B)

    deg_parts = _sc_degree(row2d)

    sig1_inv = 1.0 / (sigma1 * sigma1)
    sig2_inv = 1.0 / (sigma2 * sigma2)
    b1r = b1.reshape(1, F)
    b2pad = jnp.pad(b2, (0, 64 - NCLS)).reshape(1, 64)
    w2pad = jnp.pad(W2, ((0, 0), (0, 64 - NCLS)))

    t0s, t1s = _tc_tables1(deg_parts, x)
    sp1 = _scan_scatter_fs(t0s, t1s, row2d, col2d, 68)
    y1ps = _tc_mask_mm(sig1_inv, W1, deg_parts, x, sp1)
    zp2 = _scan_scatter_es(y1ps, row2d, col2d, 32)
    h, t0bs, t1bs = _tc_relu_tables(b1r, deg_parts, zp2, y1ps)

    sp3 = _scan_scatter_fs(t0bs, t1bs, row2d, col2d, 64)
    y2ps = _tc_mask_mm2(sig2_inv, w2pad, deg_parts, h, sp3, sp1)
    zp4 = _scan_scatter_es(y2ps, row2d, col2d, 32)
    out = _tc_final(b2pad, deg_parts, zp4, y2ps)
    return out[:, :NCLS]


# speedup vs baseline: 6.5256x; 1.1519x over previous
import functools

import jax
import jax.numpy as jnp
from jax import lax
from jax.experimental import pallas as pl
from jax.experimental.pallas import tpu as pltpu
from jax.experimental.pallas import tpu_sc as plsc

N = 10000
E = 320000
F = 128
NCLS = 40
NC = 2
NS = 16
B = 128
NW = NC * NS

E_PAD = 327680
N_ACC = 10112
RPT = N_ACC // NS

_ROWBLK = 1000
_GRID = N // _ROWBLK

_SC_PARAMS = pltpu.CompilerParams(use_tc_tiling_on_sc=False)


def _mesh():
    return plsc.VectorSubcoreMesh(
        core_axis_name="c", subcore_axis_name="s", num_cores=NC, num_subcores=NS)


_NB_DEG = E_PAD // (NW * B)


_DEGC = 8


def _sc_degree_body(row2d, ones_hbm, zeros_hbm, out_hbm,
                    rowv, onesv, stage, acc, sem):
    c = lax.axis_index("c")
    s = lax.axis_index("s")
    wid = s * NC + c
    pltpu.sync_copy(row2d.at[pl.ds(wid * _NB_DEG, _NB_DEG), :], rowv)
    pltpu.sync_copy(ones_hbm, onesv)
    pltpu.sync_copy(zeros_hbm.at[pl.ds(s * RPT, RPT), :], stage)
    pltpu.sync_copy(stage, acc.at[pl.ds(s * RPT, RPT), :])
    plsc.subcore_barrier()

    def body(j, carry):
        pltpu.sync_copy(onesv, acc.at[rowv.at[j]], add=True)
        return carry

    lax.fori_loop(0, _NB_DEG, body, 0)
    plsc.subcore_barrier()
    pltpu.sync_copy(acc.at[pl.ds(s * RPT, RPT), :], stage)
    pltpu.sync_copy(stage, out_hbm.at[c, pl.ds(s * RPT, RPT), :])


def _sc_degree(row2d):
    ones = jnp.ones((B, _DEGC), jnp.float32)
    zeros = jnp.zeros((N_ACC, _DEGC), jnp.float32)
    k = pl.kernel(
        _sc_degree_body,
        out_type=jax.ShapeDtypeStruct((NC, N_ACC, _DEGC), jnp.float32),
        mesh=_mesh(),
        scratch_types=[
            pltpu.VMEM((_NB_DEG, B), jnp.int32),
            pltpu.VMEM((B, _DEGC), jnp.float32),
            pltpu.VMEM((RPT, _DEGC), jnp.float32),
            pltpu.VMEM_SHARED((N_ACC, _DEGC), jnp.float32),
            pltpu.SemaphoreType.DMA,
        ],
        compiler_params=_SC_PARAMS,
    )
    return k(row2d, ones, zeros)


def _zero_acc(s, zeros_hbm, buf0, acc):
    pltpu.sync_copy(zeros_hbm, buf0)
    off = 0
    while off < RPT:
        rows = min(B, RPT - off)
        pltpu.sync_copy(buf0.at[pl.ds(0, rows), :],
                        acc.at[pl.ds(s * RPT + off, rows), :])
        off += rows


def _write_out(c, s, buf0, acc, out_hbm):
    off = 0
    while off < RPT:
        rows = min(B, RPT - off)
        pltpu.sync_copy(acc.at[pl.ds(s * RPT + off, rows), :],
                        buf0.at[pl.ds(0, rows), :])
        pltpu.sync_copy(buf0.at[pl.ds(0, rows), :],
                        out_hbm.at[c, pl.ds(s * RPT + off, rows), :])
        off += rows


def _write_out_k(c, s, k, buf0, acc, out_hbm):
    off = 0
    while off < RPT:
        rows = min(B, RPT - off)
        pltpu.sync_copy(acc.at[pl.ds(s * RPT + off, rows), :],
                        buf0.at[pl.ds(0, rows), :])
        pltpu.sync_copy(buf0.at[pl.ds(0, rows), :],
                        out_hbm.at[k, c, pl.ds(s * RPT + off, rows), :])
        off += rows


_NBUF = 4


def _scatter_loop(table, nb, rowv, colv, bufs, acc, gsems, ssems):
    for b in range(_NBUF):
        pltpu.async_copy(table.at[colv.at[b]], bufs[b], gsems[b])

    def body(j4, carry):
        base = _NBUF * j4
        for b in range(_NBUF):
            pltpu.make_async_copy(table.at[colv.at[base + b]],
                                  bufs[b], gsems[b]).wait()
            pltpu.async_copy(bufs[b], acc.at[rowv.at[base + b]],
                             ssems[b], add=True)
        for b in range(_NBUF):
            nxt = base + _NBUF + b

            @pl.when(nxt < nb)
            def _(b=b, nxt=nxt):
                pltpu.make_async_copy(bufs[b], acc.at[rowv.at[nxt - _NBUF]],
                                      ssems[b]).wait()
                pltpu.async_copy(table.at[colv.at[nxt]], bufs[b], gsems[b])
        return carry

    lax.fori_loop(0, nb // _NBUF, body, 0)
    for b in range(_NBUF):
        pltpu.make_async_copy(bufs[b], acc.at[rowv.at[nb - _NBUF + b]],
                              ssems[b]).wait()


def _scatter_body_fs(nb, nk, t0, t1, row2d, col2d, zeros_hbm, out_hbm,
                     rowv, colv, b0, b1, b2, b3, acc,
                     g0, g1, g2, g3, s0, s1, s2, s3):
    c = lax.axis_index("c")
    s = lax.axis_index("s")
    bufs, gsems, ssems = (b0, b1, b2, b3), (g0, g1, g2, g3), (s0, s1, s2, s3)
    pltpu.sync_copy(row2d.at[pl.ds(s * nb, nb), :], rowv)
    pltpu.sync_copy(col2d.at[pl.ds(s * nb, nb), :], colv)
    for k in range(nk):
        _zero_acc(s, zeros_hbm, b0, acc)
        plsc.subcore_barrier()

        @pl.when(c == 0)
        def _(k=k):
            _scatter_loop(t0.at[k], nb, rowv, colv, bufs, acc, gsems, ssems)

        @pl.when(c == 1)
        def _(k=k):
            _scatter_loop(t1.at[k], nb, rowv, colv, bufs, acc, gsems, ssems)

        plsc.subcore_barrier()
        _write_out_k(c, s, k, b0, acc, out_hbm)
        plsc.subcore_barrier()


def _scatter_body_es(nb, nk, t0, row2d, col2d, zeros_hbm, out_hbm,
                     rowv, colv, b0, b1, b2, b3, acc,
                     g0, g1, g2, g3, s0, s1, s2, s3):
    c = lax.axis_index("c")
    s = lax.axis_index("s")
    bufs, gsems, ssems = (b0, b1, b2, b3), (g0, g1, g2, g3), (s0, s1, s2, s3)
    wid = s * NC + c
    pltpu.sync_copy(row2d.at[pl.ds(wid * nb, nb), :], rowv)
    pltpu.sync_copy(col2d.at[pl.ds(wid * nb, nb), :], colv)
    for k in range(nk):
        _zero_acc(s, zeros_hbm, b0, acc)
        plsc.subcore_barrier()
        _scatter_loop(t0.at[k], nb, rowv, colv, bufs, acc, gsems, ssems)
        plsc.subcore_barrier()
        _write_out_k(c, s, k, b0, acc, out_hbm)
        plsc.subcore_barrier()


def _sc_scatter(ts, row2d, col2d, feature_split):
    nb = E_PAD // ((NS if feature_split else NW) * B)
    tables = ts if feature_split else (ts,)
    nk, _, ncols = tables[0].shape
    body = functools.partial(
        _scatter_body_fs if feature_split else _scatter_body_es, nb, nk)
    zeros = jnp.zeros((B, ncols), jnp.float32)
    kern = pl.kernel(
        body,
        out_type=jax.ShapeDtypeStruct((nk, NC, N_ACC, ncols), jnp.float32),
        mesh=_mesh(),
        scratch_types=[
            pltpu.VMEM((nb, B), jnp.int32),
            pltpu.VMEM((nb, B), jnp.int32),
        ]
        + [pltpu.VMEM((B, ncols), jnp.float32)] * _NBUF
        + [pltpu.VMEM_SHARED((N_ACC, ncols), jnp.float32)]
        + [pltpu.SemaphoreType.DMA] * (2 * _NBUF),
        compiler_params=_SC_PARAMS,
    )
    return kern(*tables, row2d, col2d, zeros)


def _dinv_of(pa_ref, pb_ref):
    deg = pa_ref[0, :, 0:1] + pb_ref[0, :, 0:1] + 1.0
    return lax.rsqrt(deg)


_DEG_SPEC_A = pl.BlockSpec((1, _ROWBLK, _DEGC), lambda i: (0, i, 0))
_DEG_SPEC_B = pl.BlockSpec((1, _ROWBLK, _DEGC), lambda i: (1, i, 0))


def _row_spec(ncols):
    return pl.BlockSpec((_ROWBLK, ncols), lambda i: (i, 0))


def _stack_spec(k, ncols):
    return pl.BlockSpec((1, _ROWBLK, ncols),
                        functools.partial(lambda k_, i: (k_, i, 0), k))


def _piece_spec(k, c, ncols):
    return pl.BlockSpec((1, 1, _ROWBLK, ncols),
                        functools.partial(lambda k_, c_, i: (k_, c_, i, 0), k, c))


def _full_spec(r, c):
    return pl.BlockSpec((r, c), lambda i: (0, 0))


def _tc_tables1_body(pa, pb, x_ref, t0_ref, t1_ref, d4_ref):
    dinv = _dinv_of(pa, pb)
    x = x_ref[...]
    u1 = dinv * x
    u2 = u1 * x

    def chunk(base, k):
        lo = base + 32 * k
        return jnp.concatenate([u1[:, lo:lo + 32], u2[:, lo:lo + 32]], axis=1)

    t0_ref[...] = jnp.stack([chunk(0, 0), chunk(0, 1)])
    t1_ref[...] = jnp.stack([chunk(64, 0), chunk(64, 1)])
    d4_ref[...] = (dinv + jnp.zeros((_ROWBLK, 4), jnp.float32))[None]


def _tc_tables1(deg_parts, x):
    return pl.pallas_call(
        _tc_tables1_body,
        grid=(_GRID,),
        in_specs=[_DEG_SPEC_A, _DEG_SPEC_B, _row_spec(F)],
        out_specs=[pl.BlockSpec((2, _ROWBLK, 64), lambda i: (0, i, 0))] * 2
        + [pl.BlockSpec((1, _ROWBLK, 4), lambda i: (0, i, 0))],
        out_shape=[jax.ShapeDtypeStruct((2, N, 64), jnp.float32)] * 2
        + [jax.ShapeDtypeStruct((1, N, 4), jnp.float32)],
    )(deg_parts, deg_parts, x)


def _tc_mask_mm_body(sig2_inv_ref, w_ref, pa, pb, x_ref,
                     p00, p01, p10, p11, s0a, s0b, out_ref):
    dinv = _dinv_of(pa, pb)
    x = x_ref[...]
    s1 = jnp.concatenate([p00[0, 0, :, 0:32], p10[0, 0, :, 0:32],
                          p01[0, 0, :, 0:32], p11[0, 0, :, 0:32]], axis=1)
    s2 = jnp.concatenate([p00[0, 0, :, 32:64], p10[0, 0, :, 32:64],
                          p01[0, 0, :, 32:64], p11[0, 0, :, 32:64]], axis=1)
    s0 = s0a[0, 0, :, 0:1] + s0b[0, 0, :, 0:1]
    bracket = s2 - 2.0 * x * s1 + x * x * s0
    mask = jnp.exp(-(dinv * dinv * dinv) * bracket * sig2_inv_ref[...])
    y = jnp.dot(x * mask, w_ref[...], preferred_element_type=jnp.float32)
    dy = dinv * y
    out_ref[...] = jnp.stack([dy[:, 32 * k:32 * k + 32] for k in range(4)])


def _tc_mask_mm(sig2_inv, w, deg_parts, x, sp1, s0p):
    return pl.pallas_call(
        _tc_mask_mm_body,
        grid=(_GRID,),
        in_specs=[_full_spec(1, F), _full_spec(F, F),
                  _DEG_SPEC_A, _DEG_SPEC_B, _row_spec(F),
                  _piece_spec(0, 0, 64), _piece_spec(0, 1, 64),
                  _piece_spec(1, 0, 64), _piece_spec(1, 1, 64),
                  _piece_spec(0, 0, 4), _piece_spec(0, 1, 4)],
        out_specs=pl.BlockSpec((4, _ROWBLK, 32), lambda i: (0, i, 0)),
        out_shape=jax.ShapeDtypeStruct((4, N, 32), jnp.float32),
    )(sig2_inv, w, deg_parts, deg_parts, x, sp1, sp1, sp1, sp1, s0p, s0p)


def _tc_relu_tables_body(b1_ref, pa, pb, z00, z01, z10, z11, z20, z21,
                         z30, z31, y1p_ref, h_ref, t0_ref, t1_ref):
    dinv = _dinv_of(pa, pb)
    zs = [z00[0, 0] + z01[0, 0], z10[0, 0] + z11[0, 0],
          z20[0, 0] + z21[0, 0], z30[0, 0] + z31[0, 0]]
    z = jnp.concatenate(zs, axis=1)
    y1p = jnp.concatenate([y1p_ref[k] for k in range(4)], axis=1)
    h = dinv * (z + y1p) + b1_ref[...]
    h = jnp.maximum(h, 0.0)
    h_ref[...] = h
    u1 = dinv * h
    u2 = u1 * h

    def chunk(base, k):
        lo = base + 32 * k
        return jnp.concatenate([u1[:, lo:lo + 32], u2[:, lo:lo + 32]], axis=1)

    t0_ref[...] = jnp.stack([chunk(0, k) for k in range(2)])
    t1_ref[...] = jnp.stack([chunk(64, k) for k in range(2)])


def _tc_relu_tables(b1, deg_parts, zp2, y1ps):
    return pl.pallas_call(
        _tc_relu_tables_body,
        grid=(_GRID,),
        in_specs=[_full_spec(1, F), _DEG_SPEC_A, _DEG_SPEC_B]
        + [_piece_spec(k, c, 32) for k in range(4) for c in range(2)]
        + [pl.BlockSpec((4, _ROWBLK, 32), lambda i: (0, i, 0))],
        out_specs=[_row_spec(F)]
        + [pl.BlockSpec((2, _ROWBLK, 64), lambda i: (0, i, 0))] * 2,
        out_shape=[jax.ShapeDtypeStruct((N, F), jnp.float32)]
        + [jax.ShapeDtypeStruct((2, N, 64), jnp.float32)] * 2,
    )(b1, deg_parts, deg_parts, *([zp2] * 8), y1ps)


def _tc_mask_mm2_body(sig2_inv_ref, w_ref, pa, pb, h_ref, s0a, s0b,
                      t00, t01, t10, t11, out_ref):
    dinv = _dinv_of(pa, pb)
    h = h_ref[...]
    ts = [t00, t10, t01, t11]
    s1 = jnp.concatenate([t[0, 0, :, 0:32] for t in ts], axis=1)
    s2 = jnp.concatenate([t[0, 0, :, 32:64] for t in ts], axis=1)
    s0 = s0a[0, 0, :, 0:1] + s0b[0, 0, :, 0:1]
    bracket = s2 - 2.0 * h * s1 + h * h * s0
    mask = jnp.exp(-(dinv * dinv * dinv) * bracket * sig2_inv_ref[...])
    y = jnp.dot(h * mask, w_ref[...], preferred_element_type=jnp.float32)
    dy = dinv * y
    out_ref[...] = jnp.stack([dy[:, 0:32], dy[:, 32:64]])


def _tc_mask_mm2(sig2_inv, wpad, deg_parts, h, sp3, s0p):
    return pl.pallas_call(
        _tc_mask_mm2_body,
        grid=(_GRID,),
        in_specs=[_full_spec(1, F), _full_spec(F, 64),
                  _DEG_SPEC_A, _DEG_SPEC_B, _row_spec(F),
                  _piece_spec(0, 0, 4), _piece_spec(0, 1, 4)]
        + [_piece_spec(k, c, 64) for k in range(2) for c in range(2)],
        out_specs=pl.BlockSpec((2, _ROWBLK, 32), lambda i: (0, i, 0)),
        out_shape=jax.ShapeDtypeStruct((2, N, 32), jnp.float32),
    )(sig2_inv, wpad, deg_parts, deg_parts, h, s0p, s0p, *([sp3] * 4))


def _tc_final_body(b2_ref, pa, pb, z00, z01, z10, z11, y2p_ref, out_ref):
    dinv = _dinv_of(pa, pb)
    zs = [z00[0, 0] + z01[0, 0], z10[0, 0] + z11[0, 0]]
    z = jnp.concatenate(zs, axis=1)
    y2p = jnp.concatenate([y2p_ref[0], y2p_ref[1]], axis=1)
    logits = dinv * (z + y2p) + b2_ref[...]
    colid = lax.broadcasted_iota(jnp.int32, (_ROWBLK, 64), 1)
    valid = colid < NCLS
    neg = jnp.full_like(logits, -jnp.inf)
    m = jnp.max(jnp.where(valid, logits, neg), axis=1, keepdims=True)
    e = jnp.where(valid, jnp.exp(logits - m), 0.0)
    lse = jnp.log(jnp.sum(e, axis=1, keepdims=True))
    out_ref[...] = logits - m - lse


def _tc_final(b2pad, deg_parts, zp4, y2ps):
    return pl.pallas_call(
        _tc_final_body,
        grid=(_GRID,),
        in_specs=[_full_spec(1, 64), _DEG_SPEC_A, _DEG_SPEC_B]
        + [_piece_spec(k, c, 32) for k in range(2) for c in range(2)]
        + [pl.BlockSpec((2, _ROWBLK, 32), lambda i: (0, i, 0))],
        out_specs=_row_spec(64),
        out_shape=jax.ShapeDtypeStruct((N, 64), jnp.float32),
    )(b2pad, deg_parts, deg_parts, *([zp4] * 4), y2ps)


def kernel(x, edge_index, sigma1, W1, b1, sigma2, W2, b2):
    row = edge_index[0]
    col = edge_index[1]
    pad_r = jnp.full((E_PAD - E,), N, jnp.int32)
    pad_c = jnp.zeros((E_PAD - E,), jnp.int32)
    row2d = jnp.concatenate([row, pad_r]).reshape(E_PAD // B, B)
    col2d = jnp.concatenate([col, pad_c]).reshape(E_PAD // B, B)

    deg_parts = _sc_degree(row2d)

    sig1_inv = 1.0 / (sigma1 * sigma1)
    sig2_inv = 1.0 / (sigma2 * sigma2)
    b1r = b1.reshape(1, F)
    b2pad = jnp.pad(b2, (0, 64 - NCLS)).reshape(1, 64)
    w2pad = jnp.pad(W2, ((0, 0), (0, 64 - NCLS)))

    t0s, t1s, d4s = _tc_tables1(deg_parts, x)
    s0p = _sc_scatter(d4s, row2d, col2d, feature_split=False)
    sp1 = _sc_scatter((t0s, t1s), row2d, col2d, feature_split=True)
    y1ps = _tc_mask_mm(sig1_inv, W1, deg_parts, x, sp1, s0p)
    zp2 = _sc_scatter(y1ps, row2d, col2d, feature_split=False)
    h, t0bs, t1bs = _tc_relu_tables(b1r, deg_parts, zp2, y1ps)

    sp3 = _sc_scatter((t0bs, t1bs), row2d, col2d, feature_split=True)
    y2ps = _tc_mask_mm2(sig2_inv, w2pad, deg_parts, h, sp3, s0p)
    zp4 = _sc_scatter(y2ps, row2d, col2d, feature_split=False)
    out = _tc_final(b2pad, deg_parts, zp4, y2ps)
    return out[:, :NCLS]


# speedup vs baseline: 6.6493x; 1.0190x over previous
import functools

import jax
import jax.numpy as jnp
from jax import lax
from jax.experimental import pallas as pl
from jax.experimental.pallas import tpu as pltpu
from jax.experimental.pallas import tpu_sc as plsc

N = 10000
E = 320000
F = 128
NCLS = 40
NC = 2
NS = 16
B = 128
NW = NC * NS

E_PAD = 327680
N_ACC = 10112
RPT = N_ACC // NS

_ROWBLK = 1000
_GRID = N // _ROWBLK

_SC_PARAMS = pltpu.CompilerParams(use_tc_tiling_on_sc=False)


def _mesh():
    return plsc.VectorSubcoreMesh(
        core_axis_name="c", subcore_axis_name="s", num_cores=NC, num_subcores=NS)


_NB_DEG = E_PAD // (NW * B)


_DEGC = 8


def _sc_degree_body(row2d, ones_hbm, zeros_hbm, out_hbm,
                    rowv, onesv, stage, acc, sem):
    c = lax.axis_index("c")
    s = lax.axis_index("s")
    wid = s * NC + c
    pltpu.sync_copy(row2d.at[pl.ds(wid * _NB_DEG, _NB_DEG), :], rowv)
    pltpu.sync_copy(ones_hbm, onesv)
    pltpu.sync_copy(zeros_hbm.at[pl.ds(s * RPT, RPT), :], stage)
    pltpu.sync_copy(stage, acc.at[pl.ds(s * RPT, RPT), :])
    plsc.subcore_barrier()

    def body(j, carry):
        pltpu.sync_copy(onesv, acc.at[rowv.at[j]], add=True)
        return carry

    lax.fori_loop(0, _NB_DEG, body, 0)
    plsc.subcore_barrier()
    pltpu.sync_copy(acc.at[pl.ds(s * RPT, RPT), :], stage)
    pltpu.sync_copy(stage, out_hbm.at[c, pl.ds(s * RPT, RPT), :])


def _sc_degree(row2d):
    ones = jnp.ones((B, _DEGC), jnp.float32)
    zeros = jnp.zeros((N_ACC, _DEGC), jnp.float32)
    k = pl.kernel(
        _sc_degree_body,
        out_type=jax.ShapeDtypeStruct((NC, N_ACC, _DEGC), jnp.float32),
        mesh=_mesh(),
        scratch_types=[
            pltpu.VMEM((_NB_DEG, B), jnp.int32),
            pltpu.VMEM((B, _DEGC), jnp.float32),
            pltpu.VMEM((RPT, _DEGC), jnp.float32),
            pltpu.VMEM_SHARED((N_ACC, _DEGC), jnp.float32),
            pltpu.SemaphoreType.DMA,
        ],
        compiler_params=_SC_PARAMS,
    )
    return k(row2d, ones, zeros)


def _zero_acc(s, zeros_hbm, buf0, acc):
    pltpu.sync_copy(zeros_hbm, buf0)
    off = 0
    while off < RPT:
        rows = min(B, RPT - off)
        pltpu.sync_copy(buf0.at[pl.ds(0, rows), :],
                        acc.at[pl.ds(s * RPT + off, rows), :])
        off += rows


def _write_out(c, s, buf0, acc, out_hbm):
    off = 0
    while off < RPT:
        rows = min(B, RPT - off)
        pltpu.sync_copy(acc.at[pl.ds(s * RPT + off, rows), :],
                        buf0.at[pl.ds(0, rows), :])
        pltpu.sync_copy(buf0.at[pl.ds(0, rows), :],
                        out_hbm.at[c, pl.ds(s * RPT + off, rows), :])
        off += rows


def _write_out_k(c, s, k, buf0, acc, out_hbm):
    off = 0
    while off < RPT:
        rows = min(B, RPT - off)
        pltpu.sync_copy(acc.at[pl.ds(s * RPT + off, rows), :],
                        buf0.at[pl.ds(0, rows), :])
        pltpu.sync_copy(buf0.at[pl.ds(0, rows), :],
                        out_hbm.at[k, c, pl.ds(s * RPT + off, rows), :])
        off += rows


_NBUF = 4


def _scatter_loop(table, nb, rowv, colv, bufs, acc, gsems, ssems):
    for b in range(_NBUF):
        pltpu.async_copy(table.at[colv.at[b]], bufs[b], gsems[b])

    def body(j4, carry):
        base = _NBUF * j4
        for b in range(_NBUF):
            pltpu.make_async_copy(table.at[colv.at[base + b]],
                                  bufs[b], gsems[b]).wait()
            pltpu.async_copy(bufs[b], acc.at[rowv.at[base + b]],
                             ssems[b], add=True)
        for b in range(_NBUF):
            nxt = base + _NBUF + b

            @pl.when(nxt < nb)
            def _(b=b, nxt=nxt):
                pltpu.make_async_copy(bufs[b], acc.at[rowv.at[nxt - _NBUF]],
                                      ssems[b]).wait()
                pltpu.async_copy(table.at[colv.at[nxt]], bufs[b], gsems[b])
        return carry

    lax.fori_loop(0, nb // _NBUF, body, 0)
    for b in range(_NBUF):
        pltpu.make_async_copy(bufs[b], acc.at[rowv.at[nb - _NBUF + b]],
                              ssems[b]).wait()


def _scatter_body(nb, nk, feature_split, t0, t1, row2d, col2d, zeros_hbm,
                  out_hbm, rowv, colv, b0, b1, b2, b3, acc,
                  g0, g1, g2, g3, s0, s1, s2, s3):
    c = lax.axis_index("c")
    s = lax.axis_index("s")
    bufs, gsems, ssems = (b0, b1, b2, b3), (g0, g1, g2, g3), (s0, s1, s2, s3)
    chunk = s if feature_split else s * NC + c
    pltpu.sync_copy(row2d.at[pl.ds(chunk * nb, nb), :], rowv)
    pltpu.sync_copy(col2d.at[pl.ds(chunk * nb, nb), :], colv)
    for k in range(nk):
        _zero_acc(s, zeros_hbm, b0, acc)
        plsc.subcore_barrier()

        @pl.when(c == 0)
        def _(k=k):
            _scatter_loop(t0.at[k], nb, rowv, colv, bufs, acc, gsems, ssems)

        @pl.when(c == 1)
        def _(k=k):
            _scatter_loop(t1.at[k], nb, rowv, colv, bufs, acc, gsems, ssems)

        plsc.subcore_barrier()
        _write_out_k(c, s, k, b0, acc, out_hbm)
        plsc.subcore_barrier()


def _sc_scatter(ts, row2d, col2d, feature_split):
    nb = E_PAD // ((NS if feature_split else NW) * B)
    nk, _, ncols = ts[0].shape
    body = functools.partial(_scatter_body, nb, nk, feature_split)
    zeros = jnp.zeros((B, ncols), jnp.float32)
    kern = pl.kernel(
        body,
        out_type=jax.ShapeDtypeStruct((nk, NC, N_ACC, ncols), jnp.float32),
        mesh=_mesh(),
        scratch_types=[
            pltpu.VMEM((nb, B), jnp.int32),
            pltpu.VMEM((nb, B), jnp.int32),
        ]
        + [pltpu.VMEM((B, ncols), jnp.float32)] * _NBUF
        + [pltpu.VMEM_SHARED((N_ACC, ncols), jnp.float32)]
        + [pltpu.SemaphoreType.DMA] * (2 * _NBUF),
        compiler_params=_SC_PARAMS,
    )
    return kern(ts[0], ts[1], row2d, col2d, zeros)


def _dinv_of(pa_ref, pb_ref):
    deg = pa_ref[0, :, 0:1] + pb_ref[0, :, 0:1] + 1.0
    return lax.rsqrt(deg)


_DEG_SPEC_A = pl.BlockSpec((1, _ROWBLK, _DEGC), lambda i: (0, i, 0))
_DEG_SPEC_B = pl.BlockSpec((1, _ROWBLK, _DEGC), lambda i: (1, i, 0))


def _row_spec(ncols):
    return pl.BlockSpec((_ROWBLK, ncols), lambda i: (i, 0))


def _stack_spec(k, ncols):
    return pl.BlockSpec((1, _ROWBLK, ncols),
                        functools.partial(lambda k_, i: (k_, i, 0), k))


def _piece_spec(k, c, ncols):
    return pl.BlockSpec((1, 1, _ROWBLK, ncols),
                        functools.partial(lambda k_, c_, i: (k_, c_, i, 0), k, c))


def _full_spec(r, c):
    return pl.BlockSpec((r, c), lambda i: (0, 0))


def _tc_tables1_body(pa, pb, x_ref, t0_ref, t1_ref, d4_ref, d4b_ref):
    dinv = _dinv_of(pa, pb)
    x = x_ref[...]
    u1 = dinv * x
    u2 = u1 * x

    def chunk(base, k):
        lo = base + 32 * k
        return jnp.concatenate([u1[:, lo:lo + 32], u2[:, lo:lo + 32]], axis=1)

    t0_ref[...] = jnp.stack([chunk(0, 0), chunk(0, 1)])
    t1_ref[...] = jnp.stack([chunk(64, 0), chunk(64, 1)])
    d4 = (dinv + jnp.zeros((_ROWBLK, 4), jnp.float32))[None]
    d4_ref[...] = d4
    d4b_ref[...] = d4


def _tc_tables1(deg_parts, x):
    return pl.pallas_call(
        _tc_tables1_body,
        grid=(_GRID,),
        in_specs=[_DEG_SPEC_A, _DEG_SPEC_B, _row_spec(F)],
        out_specs=[pl.BlockSpec((2, _ROWBLK, 64), lambda i: (0, i, 0))] * 2
        + [pl.BlockSpec((1, _ROWBLK, 4), lambda i: (0, i, 0))] * 2,
        out_shape=[jax.ShapeDtypeStruct((2, N, 64), jnp.float32)] * 2
        + [jax.ShapeDtypeStruct((1, N, 4), jnp.float32)] * 2,
    )(deg_parts, deg_parts, x)


def _tc_mask_mm_body(sig2_inv_ref, w_ref, pa, pb, x_ref,
                     p00, p01, p10, p11, s0a, s0b, out_ref, outb_ref):
    dinv = _dinv_of(pa, pb)
    x = x_ref[...]
    s1 = jnp.concatenate([p00[0, 0, :, 0:32], p10[0, 0, :, 0:32],
                          p01[0, 0, :, 0:32], p11[0, 0, :, 0:32]], axis=1)
    s2 = jnp.concatenate([p00[0, 0, :, 32:64], p10[0, 0, :, 32:64],
                          p01[0, 0, :, 32:64], p11[0, 0, :, 32:64]], axis=1)
    s0 = s0a[0, 0, :, 0:1] + s0b[0, 0, :, 0:1]
    bracket = s2 - 2.0 * x * s1 + x * x * s0
    mask = jnp.exp(-(dinv * dinv * dinv) * bracket * sig2_inv_ref[...])
    y = jnp.dot(x * mask, w_ref[...], preferred_element_type=jnp.float32)
    dy = dinv * y
    stk = jnp.stack([dy[:, 32 * k:32 * k + 32] for k in range(4)])
    out_ref[...] = stk
    outb_ref[...] = stk


def _tc_mask_mm(sig2_inv, w, deg_parts, x, sp1, s0p):
    return pl.pallas_call(
        _tc_mask_mm_body,
        grid=(_GRID,),
        in_specs=[_full_spec(1, F), _full_spec(F, F),
                  _DEG_SPEC_A, _DEG_SPEC_B, _row_spec(F),
                  _piece_spec(0, 0, 64), _piece_spec(0, 1, 64),
                  _piece_spec(1, 0, 64), _piece_spec(1, 1, 64),
                  _piece_spec(0, 0, 4), _piece_spec(0, 1, 4)],
        out_specs=[pl.BlockSpec((4, _ROWBLK, 32), lambda i: (0, i, 0))] * 2,
        out_shape=[jax.ShapeDtypeStruct((4, N, 32), jnp.float32)] * 2,
    )(sig2_inv, w, deg_parts, deg_parts, x, sp1, sp1, sp1, sp1, s0p, s0p)


def _tc_relu_tables_body(b1_ref, pa, pb, z00, z01, z10, z11, z20, z21,
                         z30, z31, y1p_ref, h_ref, t0_ref, t1_ref):
    dinv = _dinv_of(pa, pb)
    zs = [z00[0, 0] + z01[0, 0], z10[0, 0] + z11[0, 0],
          z20[0, 0] + z21[0, 0], z30[0, 0] + z31[0, 0]]
    z = jnp.concatenate(zs, axis=1)
    y1p = jnp.concatenate([y1p_ref[k] for k in range(4)], axis=1)
    h = dinv * (z + y1p) + b1_ref[...]
    h = jnp.maximum(h, 0.0)
    h_ref[...] = h
    u1 = dinv * h
    u2 = u1 * h

    def chunk(base, k):
        lo = base + 32 * k
        return jnp.concatenate([u1[:, lo:lo + 32], u2[:, lo:lo + 32]], axis=1)

    t0_ref[...] = jnp.stack([chunk(0, k) for k in range(2)])
    t1_ref[...] = jnp.stack([chunk(64, k) for k in range(2)])


def _tc_relu_tables(b1, deg_parts, zp2, y1ps):
    return pl.pallas_call(
        _tc_relu_tables_body,
        grid=(_GRID,),
        in_specs=[_full_spec(1, F), _DEG_SPEC_A, _DEG_SPEC_B]
        + [_piece_spec(k, c, 32) for k in range(4) for c in range(2)]
        + [pl.BlockSpec((4, _ROWBLK, 32), lambda i: (0, i, 0))],
        out_specs=[_row_spec(F)]
        + [pl.BlockSpec((2, _ROWBLK, 64), lambda i: (0, i, 0))] * 2,
        out_shape=[jax.ShapeDtypeStruct((N, F), jnp.float32)]
        + [jax.ShapeDtypeStruct((2, N, 64), jnp.float32)] * 2,
    )(b1, deg_parts, deg_parts, *([zp2] * 8), y1ps)


def _tc_mask_mm2_body(sig2_inv_ref, w_ref, pa, pb, h_ref, s0a, s0b,
                      t00, t01, t10, t11, out_ref, outb_ref):
    dinv = _dinv_of(pa, pb)
    h = h_ref[...]
    ts = [t00, t10, t01, t11]
    s1 = jnp.concatenate([t[0, 0, :, 0:32] for t in ts], axis=1)
    s2 = jnp.concatenate([t[0, 0, :, 32:64] for t in ts], axis=1)
    s0 = s0a[0, 0, :, 0:1] + s0b[0, 0, :, 0:1]
    bracket = s2 - 2.0 * h * s1 + h * h * s0
    mask = jnp.exp(-(dinv * dinv * dinv) * bracket * sig2_inv_ref[...])
    y = jnp.dot(h * mask, w_ref[...], preferred_element_type=jnp.float32)
    dy = dinv * y
    stk = jnp.stack([dy[:, 0:32], dy[:, 32:64]])
    out_ref[...] = stk
    outb_ref[...] = stk


def _tc_mask_mm2(sig2_inv, wpad, deg_parts, h, sp3, s0p):
    return pl.pallas_call(
        _tc_mask_mm2_body,
        grid=(_GRID,),
        in_specs=[_full_spec(1, F), _full_spec(F, 64),
                  _DEG_SPEC_A, _DEG_SPEC_B, _row_spec(F),
                  _piece_spec(0, 0, 4), _piece_spec(0, 1, 4)]
        + [_piece_spec(k, c, 64) for k in range(2) for c in range(2)],
        out_specs=[pl.BlockSpec((2, _ROWBLK, 32), lambda i: (0, i, 0))] * 2,
        out_shape=[jax.ShapeDtypeStruct((2, N, 32), jnp.float32)] * 2,
    )(sig2_inv, wpad, deg_parts, deg_parts, h, s0p, s0p, *([sp3] * 4))


def _tc_final_body(b2_ref, pa, pb, z00, z01, z10, z11, y2p_ref, out_ref):
    dinv = _dinv_of(pa, pb)
    zs = [z00[0, 0] + z01[0, 0], z10[0, 0] + z11[0, 0]]
    z = jnp.concatenate(zs, axis=1)
    y2p = jnp.concatenate([y2p_ref[0], y2p_ref[1]], axis=1)
    logits = dinv * (z + y2p) + b2_ref[...]
    colid = lax.broadcasted_iota(jnp.int32, (_ROWBLK, 64), 1)
    valid = colid < NCLS
    neg = jnp.full_like(logits, -jnp.inf)
    m = jnp.max(jnp.where(valid, logits, neg), axis=1, keepdims=True)
    e = jnp.where(valid, jnp.exp(logits - m), 0.0)
    lse = jnp.log(jnp.sum(e, axis=1, keepdims=True))
    out_ref[...] = logits - m - lse


def _tc_final(b2pad, deg_parts, zp4, y2ps):
    return pl.pallas_call(
        _tc_final_body,
        grid=(_GRID,),
        in_specs=[_full_spec(1, 64), _DEG_SPEC_A, _DEG_SPEC_B]
        + [_piece_spec(k, c, 32) for k in range(2) for c in range(2)]
        + [pl.BlockSpec((2, _ROWBLK, 32), lambda i: (0, i, 0))],
        out_specs=_row_spec(64),
        out_shape=jax.ShapeDtypeStruct((N, 64), jnp.float32),
    )(b2pad, deg_parts, deg_parts, *([zp4] * 4), y2ps)


def kernel(x, edge_index, sigma1, W1, b1, sigma2, W2, b2):
    row = edge_index[0]
    col = edge_index[1]
    pad_r = jnp.full((E_PAD - E,), N, jnp.int32)
    pad_c = jnp.zeros((E_PAD - E,), jnp.int32)
    row2d = jnp.concatenate([row, pad_r]).reshape(E_PAD // B, B)
    col2d = jnp.concatenate([col, pad_c]).reshape(E_PAD // B, B)

    deg_parts = _sc_degree(row2d)

    sig1_inv = 1.0 / (sigma1 * sigma1)
    sig2_inv = 1.0 / (sigma2 * sigma2)
    b1r = b1.reshape(1, F)
    b2pad = jnp.pad(b2, (0, 64 - NCLS)).reshape(1, 64)
    w2pad = jnp.pad(W2, ((0, 0), (0, 64 - NCLS)))

    t0s, t1s, d4a, d4b = _tc_tables1(deg_parts, x)
    s0p = _sc_scatter((d4a, d4b), row2d, col2d, feature_split=False)
    sp1 = _sc_scatter((t0s, t1s), row2d, col2d, feature_split=True)
    y1pa, y1pb = _tc_mask_mm(sig1_inv, W1, deg_parts, x, sp1, s0p)
    zp2 = _sc_scatter((y1pa, y1pb), row2d, col2d, feature_split=False)
    h, t0bs, t1bs = _tc_relu_tables(b1r, deg_parts, zp2, y1pa)

    sp3 = _sc_scatter((t0bs, t1bs), row2d, col2d, feature_split=True)
    y2pa, y2pb = _tc_mask_mm2(sig2_inv, w2pad, deg_parts, h, sp3, s0p)
    zp4 = _sc_scatter((y2pa, y2pb), row2d, col2d, feature_split=False)
    out = _tc_final(b2pad, deg_parts, zp4, y2pa)
    return out[:, :NCLS]
